# baseline (speedup 1.0000x reference)
import functools

import jax
import jax.numpy as jnp
from jax import lax
from jax.experimental import pallas as pl
from jax.experimental.pallas import tpu as pltpu
from jax.experimental.pallas import tpu_sc as plsc

NC = 2
NS = 16
L = 16
NW = NC * NS

CH = 128
ROWW = 144
K = 32
RBLK = 400


def _interleave_cols(m):
  r = m.reshape(m.shape[0], CH // 32, 2, L)
  return jnp.swapaxes(r, 2, 3).reshape(m.shape[0], CH)


def _tc_tables_body(x_ref, w1_ref, t_ref):
  x = x_ref[...]
  w1 = w1_ref[...]
  c_in = x.shape[1]
  p = lax.dot_general(x, w1[:, :c_in], (((1,), (1,)), ((), ())),
                      preferred_element_type=jnp.float32)
  q = lax.dot_general(x, w1[:, c_in:], (((1,), (1,)), ((), ())),
                      preferred_element_type=jnp.float32)
  t_ref[0] = _interleave_cols(p[:, :CH])
  t_ref[1] = _interleave_cols(p[:, CH:])
  t_ref[2] = _interleave_cols(q[:, :CH])
  t_ref[3] = _interleave_cols(q[:, CH:])


def _tc_tables(x, w1):
  n, c_in = x.shape
  grid = n // RBLK
  return pl.pallas_call(
      _tc_tables_body,
      grid=(grid,),
      in_specs=[
          pl.BlockSpec((RBLK, c_in), lambda i: (i, 0)),
          pl.BlockSpec(w1.shape, lambda i: (0, 0)),
      ],
      out_specs=pl.BlockSpec((4, RBLK, CH), lambda i: (0, i, 0)),
      out_shape=jax.ShapeDtypeStruct((4, n, CH), jnp.float32),
  )(x, w1)


def _sc_body(n_nodes, e_edges,
             t_hbm, ei_hbm, ew_hbm, b1_hbm, out_hbm,
             sd_a, sd_b, sd_c, ew_a, ew_b, ew_c, gi_a, gi_b, gi_c,
             ss_a, ss_b, ss_c, pq_a, pq_b, pq_c, ho_a, ho_b, ho_c,
             b1_v, acc,
             si_a, si_b, si_c, sw_a, sw_b, sw_c, sp_a, sp_b, sp_c,
             sa_a, sa_b, sa_c):
  ept = e_edges // NS
  nch = ept // K
  npad = ((n_nodes + 8 * NS - 1) // (8 * NS)) * (8 * NS)
  slab = npad // NS

  cid = lax.axis_index("c")
  sid = lax.axis_index("s")
  base = sid * ept
  off_p = cid * n_nodes
  off_q = (2 + cid) * n_nodes

  slots = (
      (sd_a, ew_a, gi_a, ss_a, pq_a, ho_a, si_a, sw_a, sp_a, sa_a),
      (sd_b, ew_b, gi_b, ss_b, pq_b, ho_b, si_b, sw_b, sp_b, sa_b),
      (sd_c, ew_c, gi_c, ss_c, pq_c, ho_c, si_c, sw_c, sp_c, sa_c),
  )

  def idx_start(ci, s):
    sd, ew, gi, ss, pq, ho, si, sw, sp, sa = s
    eb = base + ci * K
    pltpu.async_copy(ei_hbm.at[:, pl.ds(eb, K)], sd, si)
    pltpu.async_copy(ew_hbm.at[pl.ds(eb, K)], ew.at[pl.ds(0, K)], sw)

  def idx_wait(s):
    sd, ew, gi, ss, pq, ho, si, sw, sp, sa = s
    pltpu.make_async_copy(ei_hbm.at[:, pl.ds(0, K)], sd, si).wait()
    pltpu.make_async_copy(ew_hbm.at[pl.ds(0, K)], ew.at[pl.ds(0, K)],
                          sw).wait()

  def prep(s):
    sd, ew, gi, ss, pq, ho, si, sw, sp, sa = s
    for t in range(K // L):
      sl = pl.ds(t * L, L)
      sv = sd[0, sl]
      dv = sd[1, sl]
      gi[sl] = dv + off_p
      gi[pl.ds(K + t * L, L)] = sv + off_q
      ss[sl] = dv

  def gather_start(s):
    sd, ew, gi, ss, pq, ho, si, sw, sp, sa = s
    pltpu.async_copy(t_hbm.at[gi], pq, sp)

  def gather_wait(s):
    sd, ew, gi, ss, pq, ho, si, sw, sp, sa = s
    pltpu.make_async_copy(t_hbm.at[gi], pq, sp).wait()

  def compute(s):
    sd, ew, gi, ss, pq, ho, si, sw, sp, sa = s

    b1c = [b1_v[pl.ds(c * L, L)] for c in range(CH // L)]

    himask = jnp.full((L,), -65536, jnp.int32)

    def _bf16_pair(x):
      return (lax.bitcast_convert_type(x << 16, jnp.float32),
              lax.bitcast_convert_type(x & himask, jnp.float32))

    @plsc.parallel_loop(0, K, 1, unroll=4)
    def _(j):
      w = ew[pl.ds(j, L)][0]
      vs = []
      for g in range(CH // 32):
        pa, pb = _bf16_pair(pq[j, pl.ds(g * L, L)])
        qa, qb = _bf16_pair(pq[K + j, pl.ds(g * L, L)])
        vs.append((pa + qa) * w + b1c[2 * g])
        vs.append((pb + qb) * w + b1c[2 * g + 1])
      es = [jnp.exp(jnp.minimum(v, 0.0)) for v in vs]
      for c in range(CH // L):
        ho[j, pl.ds(c * L, L)] = jnp.maximum(vs[c], 0.0) + es[c]

  def scat_start(s):
    sd, ew, gi, ss, pq, ho, si, sw, sp, sa = s
    pltpu.async_copy(ho, acc.at[ss], sa, add=True)

  def scat_wait(s):
    sd, ew, gi, ss, pq, ho, si, sw, sp, sa = s
    pltpu.make_async_copy(ho, acc.at[ss], sa).wait()

  zero16 = jnp.zeros((L,), jnp.float32)
  cnt_vec = jnp.where(lax.iota(jnp.int32, L) == 0, 1.0, 0.0)

  def zrow(r, carry):
    for c9 in range(ROWW // L):
      ho_a[r, pl.ds(c9 * L, L)] = zero16
    return carry
  lax.fori_loop(0, 8, zrow, 0)

  def zslab(r, carry):
    pltpu.sync_copy(ho_a.at[pl.ds(0, 8)],
                    acc.at[pl.ds(sid * slab + r * 8, 8)])
    return carry
  lax.fori_loop(0, slab // 8, zslab, 0)

  def hrow(r, carry):
    ho_a[r, pl.ds(CH, L)] = cnt_vec
    ho_b[r, pl.ds(CH, L)] = cnt_vec
    ho_c[r, pl.ds(CH, L)] = cnt_vec
    return carry
  lax.fori_loop(0, K, hrow, 0)

  pltpu.sync_copy(b1_hbm.at[cid], b1_v)
  plsc.subcore_barrier()

  idx_start(0, slots[0])
  idx_start(1, slots[1])
  idx_start(2, slots[2])
  idx_wait(slots[0])
  prep(slots[0])
  gather_start(slots[0])
  idx_wait(slots[1])
  prep(slots[1])
  gather_start(slots[1])

  gather_wait(slots[0])
  idx_wait(slots[2])
  prep(slots[2])
  gather_start(slots[2])
  compute(slots[0])
  scat_start(slots[0])
  idx_start(3, slots[0])

  def triple(p, carry):
    for par in range(3):
      ci = 1 + 3 * p + par
      s = slots[(1 + par) % 3]
      s2 = slots[(1 + par + 2) % 3]
      gather_wait(s)
      idx_wait(s2)
      scat_wait(s2)
      prep(s2)
      gather_start(s2)
      compute(s)
      scat_start(s)
      idx_start(ci + 3, s)
    return carry
  lax.fori_loop(0, (nch - 4) // 3, triple, 0)

  ci = nch - 3
  s = slots[ci % 3]
  s2 = slots[(ci + 2) % 3]
  gather_wait(s)
  idx_wait(s2)
  scat_wait(s2)
  prep(s2)
  gather_start(s2)
  compute(s)
  scat_start(s)

  ci = nch - 2
  s = slots[ci % 3]
  s2 = slots[(ci + 2) % 3]
  gather_wait(s)
  scat_wait(s2)
  compute(s)
  scat_start(s)

  ci = nch - 1
  s = slots[ci % 3]
  s2 = slots[(ci + 2) % 3]
  gather_wait(s)
  scat_wait(s2)
  compute(s)
  scat_start(s)
  scat_wait(s)

  plsc.subcore_barrier()

  def outslab(r, carry):
    row = sid * slab + r * 8
    pltpu.sync_copy(acc.at[pl.ds(row, 8)], ho_a.at[pl.ds(0, 8)])
    pltpu.sync_copy(ho_a.at[pl.ds(0, 8)], out_hbm.at[cid, pl.ds(row, 8)])
    return carry
  lax.fori_loop(0, slab // 8, outslab, 0)


def _sc_stage(tab, ei, ew, b1t):
  n = tab.shape[0] // 4
  e = ei.shape[1]
  nch = e // NS // K
  assert e == NS * K * nch and (nch - 4) % 3 == 0 and nch >= 7
  npad = ((n + 8 * NS - 1) // (8 * NS)) * (8 * NS)
  mesh = plsc.VectorSubcoreMesh(core_axis_name="c", subcore_axis_name="s",
                                num_cores=NC, num_subcores=NS)
  sd_t = pltpu.VMEM((2, K), jnp.int32)
  ew_t = pltpu.VMEM((K + L,), jnp.float32)
  gi_t = pltpu.VMEM((2 * K,), jnp.int32)
  ss_t = pltpu.VMEM((K,), jnp.int32)
  pq_t = pltpu.VMEM((2 * K, CH // 2), jnp.int32)
  ho_t = pltpu.VMEM((K, ROWW), jnp.float32)
  fn = pl.kernel(
      functools.partial(_sc_body, n, e),
      out_type=jax.ShapeDtypeStruct((NC, npad, ROWW), jnp.float32),
      mesh=mesh,
      scratch_types=[
          sd_t, sd_t, sd_t, ew_t, ew_t, ew_t, gi_t, gi_t, gi_t,
          ss_t, ss_t, ss_t, pq_t, pq_t, pq_t, ho_t, ho_t, ho_t,
          pltpu.VMEM((CH,), jnp.float32),
          pltpu.VMEM_SHARED((npad, ROWW), jnp.float32),
      ] + [pltpu.SemaphoreType.DMA] * 12,
      compiler_params=pltpu.CompilerParams(use_tc_tiling_on_sc=False),
  )
  return fn(tab, ei, ew, b1t)


def _tc_final_body(p_ref, w2_ref, b2_ref, out_ref):
  a = p_ref[...]
  s0 = a[0]
  s1 = a[1]
  cnt = s0[:, CH:CH + 1]
  h = jnp.concatenate([s0[:, :CH], s1[:, :CH]], axis=1)
  hm = h / jnp.maximum(cnt, 1.0) - 1.0
  y = lax.dot_general(hm, w2_ref[...], (((1,), (1,)), ((), ())),
                      preferred_element_type=jnp.float32) + b2_ref[...]
  out_ref[...] = jnp.where(cnt > 0, y, 0.0)


def _tc_final(part, w2, b2, n):
  grid = n // RBLK
  return pl.pallas_call(
      _tc_final_body,
      grid=(grid,),
      in_specs=[
          pl.BlockSpec((NC, RBLK, ROWW), lambda i: (0, i, 0)),
          pl.BlockSpec(w2.shape, lambda i: (0, 0)),
          pl.BlockSpec((1, w2.shape[0]), lambda i: (0, 0)),
      ],
      out_specs=pl.BlockSpec((RBLK, w2.shape[0]), lambda i: (i, 0)),
      out_shape=jax.ShapeDtypeStruct((n, w2.shape[0]), jnp.float32),
  )(part, w2, b2.reshape(1, -1))


@jax.jit
def kernel(x, edge_index, edge_weight, W1, b1, W2, b2):
  n, c_in = x.shape
  ei = edge_index.astype(jnp.int32)

  tab = _tc_tables(x, W1)

  b1t = b1.reshape(NC, CH)
  tab_i32 = lax.bitcast_convert_type(
      tab.astype(jnp.bfloat16).reshape(4 * n, CH // 2, 2), jnp.int32)
  part = _sc_stage(tab_i32, ei, edge_weight, b1t)

  out = _tc_final(part, W2, b2, n)
  return out[None]


# speedup vs baseline: 2.6299x; 2.6299x over previous
import functools

import jax
import jax.numpy as jnp
from jax import lax
from jax.experimental import pallas as pl
from jax.experimental.pallas import tpu as pltpu
from jax.experimental.pallas import tpu_sc as plsc

NC = 2
NS = 16
L = 16
NW = NC * NS

CH = 128
ROWW = 144
K = 32
RBLK = 400


def _tc_tables_body(x_ref, w1_ref, t_ref):
  x = x_ref[...]
  w1 = w1_ref[...]
  c_in = x.shape[1]
  p = lax.dot_general(x, w1[:, :c_in], (((1,), (1,)), ((), ())),
                      preferred_element_type=jnp.float32)
  q = lax.dot_general(x, w1[:, c_in:], (((1,), (1,)), ((), ())),
                      preferred_element_type=jnp.float32)
  t_ref[0] = p[:, :CH]
  t_ref[1] = p[:, CH:]
  t_ref[2] = q[:, :CH]
  t_ref[3] = q[:, CH:]


def _tc_tables(x, w1):
  n, c_in = x.shape
  grid = n // RBLK
  return pl.pallas_call(
      _tc_tables_body,
      grid=(grid,),
      in_specs=[
          pl.BlockSpec((RBLK, c_in), lambda i: (i, 0)),
          pl.BlockSpec(w1.shape, lambda i: (0, 0)),
      ],
      out_specs=pl.BlockSpec((4, RBLK, CH), lambda i: (0, i, 0)),
      out_shape=jax.ShapeDtypeStruct((4, n, CH), jnp.float32),
  )(x, w1)


def _sc_body(n_nodes, e_edges,
             t_hbm, ei_hbm, ew_hbm, b1_hbm, out_hbm,
             sd_a, sd_b, sd_c, ew_a, ew_b, ew_c, gi_a, gi_b, gi_c,
             ss_a, ss_b, ss_c, pq_a, pq_b, pq_c, ho_a, ho_b, ho_c,
             b1_v, acc,
             si_a, si_b, si_c, sw_a, sw_b, sw_c, sp_a, sp_b, sp_c,
             sa_a, sa_b, sa_c):
  ept = e_edges // NS
  nch = ept // K
  npad = ((n_nodes + 8 * NS - 1) // (8 * NS)) * (8 * NS)
  slab = npad // NS

  cid = lax.axis_index("c")
  sid = lax.axis_index("s")
  base = sid * ept
  off_p = cid * n_nodes
  off_q = (2 + cid) * n_nodes

  slots = (
      (sd_a, ew_a, gi_a, ss_a, pq_a, ho_a, si_a, sw_a, sp_a, sa_a),
      (sd_b, ew_b, gi_b, ss_b, pq_b, ho_b, si_b, sw_b, sp_b, sa_b),
      (sd_c, ew_c, gi_c, ss_c, pq_c, ho_c, si_c, sw_c, sp_c, sa_c),
  )

  def idx_start(ci, s):
    sd, ew, gi, ss, pq, ho, si, sw, sp, sa = s
    eb = base + ci * K
    pltpu.async_copy(ei_hbm.at[:, pl.ds(eb, K)], sd, si)
    pltpu.async_copy(ew_hbm.at[pl.ds(eb, K)], ew.at[pl.ds(0, K)], sw)

  def idx_wait(s):
    sd, ew, gi, ss, pq, ho, si, sw, sp, sa = s
    pltpu.make_async_copy(ei_hbm.at[:, pl.ds(0, K)], sd, si).wait()
    pltpu.make_async_copy(ew_hbm.at[pl.ds(0, K)], ew.at[pl.ds(0, K)],
                          sw).wait()

  def prep(s):
    sd, ew, gi, ss, pq, ho, si, sw, sp, sa = s
    for t in range(K // L):
      sl = pl.ds(t * L, L)
      sv = sd[0, sl]
      dv = sd[1, sl]
      gi[sl] = dv + off_p
      gi[pl.ds(K + t * L, L)] = sv + off_q
      ss[sl] = dv

  def gather_start(s):
    sd, ew, gi, ss, pq, ho, si, sw, sp, sa = s
    pltpu.async_copy(t_hbm.at[gi], pq, sp)

  def gather_wait(s):
    sd, ew, gi, ss, pq, ho, si, sw, sp, sa = s
    pltpu.make_async_copy(t_hbm.at[gi], pq, sp).wait()

  def compute(s):
    sd, ew, gi, ss, pq, ho, si, sw, sp, sa = s

    b1c = [b1_v[pl.ds(c * L, L)] for c in range(CH // L)]

    @plsc.parallel_loop(0, K, 1, unroll=8)
    def _(j):
      w = ew[pl.ds(j, L)][0]
      vs = [(pq[j, pl.ds(c * L, L)] + pq[K + j, pl.ds(c * L, L)]) * w + b1c[c]
            for c in range(CH // L)]
      es = [jnp.exp(jnp.minimum(v, 0.0)) for v in vs]
      for c in range(CH // L):
        ho[j, pl.ds(c * L, L)] = jnp.maximum(vs[c], 0.0) + es[c]

  def scat_start(s):
    sd, ew, gi, ss, pq, ho, si, sw, sp, sa = s
    pltpu.async_copy(ho, acc.at[ss], sa, add=True)

  def scat_wait(s):
    sd, ew, gi, ss, pq, ho, si, sw, sp, sa = s
    pltpu.make_async_copy(ho, acc.at[ss], sa).wait()

  zero16 = jnp.zeros((L,), jnp.float32)
  cnt_vec = jnp.where(lax.iota(jnp.int32, L) == 0, 1.0, 0.0)

  def zrow(r, carry):
    for c9 in range(ROWW // L):
      ho_a[r, pl.ds(c9 * L, L)] = zero16
    return carry
  lax.fori_loop(0, 8, zrow, 0)

  def zslab(r, carry):
    pltpu.sync_copy(ho_a.at[pl.ds(0, 8)],
                    acc.at[pl.ds(sid * slab + r * 8, 8)])
    return carry
  lax.fori_loop(0, slab // 8, zslab, 0)

  def hrow(r, carry):
    ho_a[r, pl.ds(CH, L)] = cnt_vec
    ho_b[r, pl.ds(CH, L)] = cnt_vec
    ho_c[r, pl.ds(CH, L)] = cnt_vec
    return carry
  lax.fori_loop(0, K, hrow, 0)

  pltpu.sync_copy(b1_hbm.at[cid], b1_v)
  plsc.subcore_barrier()

  idx_start(0, slots[0])
  idx_start(1, slots[1])
  idx_start(2, slots[2])
  idx_wait(slots[0])
  prep(slots[0])
  gather_start(slots[0])
  idx_wait(slots[1])
  prep(slots[1])
  gather_start(slots[1])

  gather_wait(slots[0])
  idx_wait(slots[2])
  prep(slots[2])
  gather_start(slots[2])
  compute(slots[0])
  scat_start(slots[0])
  idx_start(3, slots[0])

  def triple(p, carry):
    for par in range(3):
      ci = 1 + 3 * p + par
      s = slots[(1 + par) % 3]
      s2 = slots[(1 + par + 2) % 3]
      gather_wait(s)
      idx_wait(s2)
      scat_wait(s2)
      prep(s2)
      gather_start(s2)
      compute(s)
      scat_start(s)
      idx_start(ci + 3, s)
    return carry
  lax.fori_loop(0, (nch - 4) // 3, triple, 0)

  ci = nch - 3
  s = slots[ci % 3]
  s2 = slots[(ci + 2) % 3]
  gather_wait(s)
  idx_wait(s2)
  scat_wait(s2)
  prep(s2)
  gather_start(s2)
  compute(s)
  scat_start(s)

  ci = nch - 2
  s = slots[ci % 3]
  s2 = slots[(ci + 2) % 3]
  gather_wait(s)
  scat_wait(s2)
  compute(s)
  scat_start(s)

  ci = nch - 1
  s = slots[ci % 3]
  s2 = slots[(ci + 2) % 3]
  gather_wait(s)
  scat_wait(s2)
  compute(s)
  scat_start(s)
  scat_wait(s)

  plsc.subcore_barrier()

  def outslab(r, carry):
    row = sid * slab + r * 8
    pltpu.sync_copy(acc.at[pl.ds(row, 8)], ho_a.at[pl.ds(0, 8)])
    pltpu.sync_copy(ho_a.at[pl.ds(0, 8)], out_hbm.at[cid, pl.ds(row, 8)])
    return carry
  lax.fori_loop(0, slab // 8, outslab, 0)


def _sc_stage(tab, ei, ew, b1t):
  n = tab.shape[0] // 4
  e = ei.shape[1]
  nch = e // NS // K
  assert e == NS * K * nch and (nch - 4) % 3 == 0 and nch >= 7
  npad = ((n + 8 * NS - 1) // (8 * NS)) * (8 * NS)
  mesh = plsc.VectorSubcoreMesh(core_axis_name="c", subcore_axis_name="s",
                                num_cores=NC, num_subcores=NS)
  sd_t = pltpu.VMEM((2, K), jnp.int32)
  ew_t = pltpu.VMEM((K + L,), jnp.float32)
  gi_t = pltpu.VMEM((2 * K,), jnp.int32)
  ss_t = pltpu.VMEM((K,), jnp.int32)
  pq_t = pltpu.VMEM((2 * K, CH), jnp.float32)
  ho_t = pltpu.VMEM((K, ROWW), jnp.float32)
  fn = pl.kernel(
      functools.partial(_sc_body, n, e),
      out_type=jax.ShapeDtypeStruct((NC, npad, ROWW), jnp.float32),
      mesh=mesh,
      scratch_types=[
          sd_t, sd_t, sd_t, ew_t, ew_t, ew_t, gi_t, gi_t, gi_t,
          ss_t, ss_t, ss_t, pq_t, pq_t, pq_t, ho_t, ho_t, ho_t,
          pltpu.VMEM((CH,), jnp.float32),
          pltpu.VMEM_SHARED((npad, ROWW), jnp.float32),
      ] + [pltpu.SemaphoreType.DMA] * 12,
      compiler_params=pltpu.CompilerParams(use_tc_tiling_on_sc=False),
  )
  return fn(tab, ei, ew, b1t)


def _tc_final_body(p_ref, w2_ref, b2_ref, out_ref):
  a = p_ref[...]
  s0 = a[0]
  s1 = a[1]
  cnt = s0[:, CH:CH + 1]
  h = jnp.concatenate([s0[:, :CH], s1[:, :CH]], axis=1)
  hm = h / jnp.maximum(cnt, 1.0) - 1.0
  y = lax.dot_general(hm, w2_ref[...], (((1,), (1,)), ((), ())),
                      preferred_element_type=jnp.float32) + b2_ref[...]
  out_ref[...] = jnp.where(cnt > 0, y, 0.0)


def _tc_final(part, w2, b2, n):
  grid = n // RBLK
  return pl.pallas_call(
      _tc_final_body,
      grid=(grid,),
      in_specs=[
          pl.BlockSpec((NC, RBLK, ROWW), lambda i: (0, i, 0)),
          pl.BlockSpec(w2.shape, lambda i: (0, 0)),
          pl.BlockSpec((1, w2.shape[0]), lambda i: (0, 0)),
      ],
      out_specs=pl.BlockSpec((RBLK, w2.shape[0]), lambda i: (i, 0)),
      out_shape=jax.ShapeDtypeStruct((n, w2.shape[0]), jnp.float32),
  )(part, w2, b2.reshape(1, -1))


@jax.jit
def kernel(x, edge_index, edge_weight, W1, b1, W2, b2):
  n, c_in = x.shape
  ei = edge_index.astype(jnp.int32)

  tab = _tc_tables(x, W1)

  b1t = b1.reshape(NC, CH)
  part = _sc_stage(tab.reshape(4 * n, CH), ei, edge_weight, b1t)

  out = _tc_final(part, W2, b2, n)
  return out[None]
